# baseline (speedup 1.0000x reference)
import functools

import jax
import jax.numpy as jnp
from jax import lax
from jax.experimental import pallas as pl
from jax.experimental.pallas import tpu as pltpu
from jax.experimental.pallas import tpu_sc as plsc

_NC = 2
_NS = 16
_L = 16
_F = 66
_D = 4
_RW = 8
_H1 = 16
_H2 = 8
_SPB = 64
_G = _SPB // _L

_OW1 = 0
_OB1 = _OW1 + _F * _D * _D * _L
_OW2 = _OB1 + _H1
_OB2 = _OW2 + _H2 * _H1
_OW3 = _OB2 + _H2
_OB3 = _OW3 + _H2
_WLEN = 17064


@functools.lru_cache(maxsize=None)
def _make_fused(B, n_idx):
    nw = _NC * _NS
    spw = B // nw
    n_chunks = spw // _SPB
    ch = _SPB * _F
    assert spw % _SPB == 0 and ch % 8 == 0 and n_chunks % 2 == 0

    mesh = plsc.VectorSubcoreMesh(
        core_axis_name="c", subcore_axis_name="s",
        num_cores=_NC, num_subcores=_NS)

    @functools.partial(
        pl.kernel,
        out_type=jax.ShapeDtypeStruct((B,), jnp.float32),
        mesh=mesh,
        scratch_types=[
            pltpu.VMEM((ch,), jnp.int32),
            pltpu.VMEM((ch,), jnp.int32),
            pltpu.VMEM((ch, _RW), jnp.float32),
            pltpu.VMEM((ch, _RW), jnp.float32),
            pltpu.VMEM((_WLEN,), jnp.float32),
            pltpu.VMEM(((_G * _H1 + _G) * _L,), jnp.float32),
            pltpu.VMEM((spw,), jnp.float32),
            pltpu.SemaphoreType.DMA((2,)),
        ],
        compiler_params=pltpu.CompilerParams(
            use_tc_tiling_on_sc=False, needs_layout_passes=False),
    )
    def fused_k(tab_hbm, idx_hbm, wpack_hbm, out_hbm,
                i0_v, i1_v, r0_v, r1_v, w_v, h1_v, out_v, gsem):
        wid = lax.axis_index("s") * _NC + lax.axis_index("c")
        sbase = wid * spw
        ibase = wid * spw * _F
        idx_bufs = (i0_v, i1_v)
        row_bufs = (r0_v, r1_v)

        pltpu.sync_copy(wpack_hbm, w_v)

        iota = lax.iota(jnp.int32, _L)
        rep = [((iota // 4) + sg * 4) * _F for sg in range(_D)]
        rowb = iota * _F
        tailb = (iota // 4) * (_D * _L) + (iota % 4) * _D
        dcol = [jnp.full((_L,), d, jnp.int32) for d in range(_D + 1)]
        zero = jnp.zeros((_L,), jnp.float32)

        def start_gather_dyn(c_off, parity):
            pltpu.sync_copy(
                idx_hbm.at[pl.ds(ibase + c_off * ch, ch)], idx_bufs[parity])
            pltpu.async_copy(
                tab_hbm.at[idx_bufs[parity]], row_bufs[parity],
                gsem.at[parity])

        def wait_gather(parity):
            pltpu.make_async_copy(
                tab_hbm.at[pl.ds(0, ch)], row_bufs[parity],
                gsem.at[parity]).wait()

        def layer1(rv):
            def g_body(g, _, rv=rv):
                goff = jnp.full((_L,), g * _L * _F, jnp.int32)
                ones = jnp.full((_L,), 1, jnp.int32)

                def f_body(f, carry):
                    accs = list(carry[:16])
                    bacc = carry[16]
                    idxs = list(carry[17:21])
                    bidx = carry[21]
                    wbase = (f * _D) * _L * _D
                    for d in range(_D):
                        xd = [plsc.load_gather(rv, [idxs[sg], dcol[d]])
                              for sg in range(_D)]
                        for jb in range(_D):
                            wv = w_v[pl.ds(_OW1 + wbase + (d * _D + jb) * _L,
                                           _L)]
                            for sg in range(_D):
                                accs[sg * _D + jb] = (
                                    accs[sg * _D + jb] + xd[sg] * wv)
                    bacc = bacc + plsc.load_gather(rv, [bidx, dcol[_D]])
                    return (tuple(accs) + (bacc,)
                            + tuple(i + ones for i in idxs) + (bidx + ones,))

                init = ((zero,) * 16 + (zero,)
                        + tuple(r + goff for r in rep) + (rowb + goff,))
                out = lax.fori_loop(0, _F, f_body, init, unroll=1)
                for sg in range(_D):
                    for jb in range(_D):
                        h1_v[pl.ds((g * _L + sg * _D + jb) * _L, _L)] = (
                            out[sg * _D + jb])
                h1_v[pl.ds((_G * _H1 + g) * _L, _L)] = out[16]
                return 0

            lax.fori_loop(0, _G, g_body, 0, unroll=1)

        def tail(out_off):
            b1v = w_v[pl.ds(_OB1, _L)]
            b2v = w_v[pl.ds(_OB2, _L)]
            w3v = w_v[pl.ds(_OW3, _L)]

            def g_body(g, _):
                h1 = []
                for jb in range(_D):
                    for u in range(_D):
                        idx = tailb + (g * (_H1 * _L) + jb * _L + u)
                        v = plsc.load_gather(h1_v, [idx])
                        h1.append(jnp.maximum(v + b1v[jb * _D + u], 0.0))
                h2 = []
                for k in range(_H2):
                    wv2 = w_v[pl.ds(_OW2 + k * _H1, _L)]
                    a = zero
                    for j in range(_H1):
                        a = a + h1[j] * wv2[j]
                    h2.append(jnp.maximum(a + b2v[k], 0.0))
                o = zero
                for k in range(_H2):
                    o = o + h2[k] * w3v[k]
                bacc = h1_v[pl.ds((_G * _H1 + g) * _L, _L)]
                o = o + w3v[_OB3 - _OW3] + bacc
                out_v[pl.ds(out_off + g * _L, _L)] = o
                return 0

            lax.fori_loop(0, _G, g_body, 0, unroll=1)

        start_gather_dyn(0, 0)
        start_gather_dyn(1, 1)

        def pair_body(p, _):
            c0 = 2 * p
            wait_gather(0)
            layer1(r0_v)

            @pl.when(p < (n_chunks // 2) - 1)
            def _():
                start_gather_dyn(c0 + 2, 0)

            tail(c0 * _SPB)
            wait_gather(1)
            layer1(r1_v)

            @pl.when(p < (n_chunks // 2) - 1)
            def _():
                start_gather_dyn(c0 + 3, 1)

            tail((c0 + 1) * _SPB)
            return 0

        lax.fori_loop(0, n_chunks // 2, pair_body, 0, unroll=1)

        pltpu.sync_copy(out_v, out_hbm.at[pl.ds(sbase, spw)])

    return fused_k


def kernel(fids_batch, emb_w, emb_b, W1, b1, W2, b2, W3, b3):
    B, F = fids_batch.shape
    V, D = emb_w.shape
    N = B * F

    tab = jnp.concatenate(
        [emb_w, emb_b[:, None], jnp.zeros((V, _RW - D - 1), jnp.float32)],
        axis=1)
    fids_flat = fids_batch.reshape(N)

    W1T = W1.T
    wrep = jnp.broadcast_to(
        W1T.reshape(F * D, _D, 1, _D), (F * D, _D, _D, _D)).reshape(-1)

    wpack = jnp.concatenate([
        wrep,
        b1,
        W2.reshape(_H2 * _H1),
        b2,
        W3.reshape(_H2),
        b3,
        jnp.zeros((_WLEN - _OB3 - 1,), jnp.float32),
    ])

    return _make_fused(B, N)(tab, fids_flat, wpack)


# speedup vs baseline: 1.2868x; 1.2868x over previous
import functools

import jax
import jax.numpy as jnp
from jax import lax
from jax.experimental import pallas as pl
from jax.experimental.pallas import tpu as pltpu
from jax.experimental.pallas import tpu_sc as plsc

_NC = 2
_NS = 16
_SEG = 2


@functools.lru_cache(maxsize=None)
def _make_gather(n_idx, row_w, n_chunks):
    nw = _NC * _NS
    per_w = n_idx // nw
    ch = per_w // n_chunks
    assert per_w % n_chunks == 0 and ch % 8 == 0

    mesh = plsc.VectorSubcoreMesh(
        core_axis_name="c", subcore_axis_name="s",
        num_cores=_NC, num_subcores=_NS)

    @functools.partial(
        pl.kernel,
        out_type=jax.ShapeDtypeStruct((n_idx, row_w), jnp.float32),
        mesh=mesh,
        scratch_types=[
            pltpu.VMEM((2, ch), jnp.int32),
            pltpu.VMEM((2, ch, row_w), jnp.float32),
            pltpu.SemaphoreType.DMA((2,)),
            pltpu.SemaphoreType.DMA((2,)),
        ],
        compiler_params=pltpu.CompilerParams(use_tc_tiling_on_sc=False),
    )
    def gather_k(tab_hbm, idx_hbm, out_hbm, idx_v, rows_v, gsem, wsem):
        wid = lax.axis_index("s") * _NC + lax.axis_index("c")
        base = wid * per_w

        def start_gather(c, b):
            pltpu.sync_copy(idx_hbm.at[pl.ds(base + c * ch, ch)], idx_v.at[b])
            return pltpu.async_copy(
                tab_hbm.at[idx_v.at[b]], rows_v.at[b], gsem.at[b])

        gathers = {0: start_gather(0, 0)}
        writes = {}
        for c in range(n_chunks):
            b = c % 2
            if c + 1 < n_chunks:
                if c - 1 >= 0:
                    writes[c - 1].wait()
                gathers[c + 1] = start_gather(c + 1, (c + 1) % 2)
            gathers[c].wait()
            writes[c] = pltpu.async_copy(
                rows_v.at[b], out_hbm.at[pl.ds(base + c * ch, ch)],
                wsem.at[b])
        writes[n_chunks - 2].wait()
        writes[n_chunks - 1].wait()

    return gather_k


def _mlp_body(x_ref, wcat_ref, b1_ref, w2_ref, b2_ref, w3_ref, b3_ref, o_ref):
    x = x_ref[...]
    y = jnp.dot(x, wcat_ref[...], preferred_element_type=jnp.float32)
    h1 = jnp.maximum(y[:, :-1] + b1_ref[...], 0.0)
    s = y[:, -1:]
    h2 = jnp.maximum(
        jnp.dot(h1, w2_ref[...], preferred_element_type=jnp.float32)
        + b2_ref[...], 0.0)
    o_ref[...] = (
        jnp.dot(h2, w3_ref[...], preferred_element_type=jnp.float32)
        + b3_ref[...] + s)


def kernel(fids_batch, emb_w, emb_b, W1, b1, W2, b2, W3, b3):
    B, F = fids_batch.shape
    V, D = emb_w.shape
    H1, IN = W1.shape
    H2 = W2.shape[0]
    RW = 8
    N = B * F

    tab = jnp.concatenate(
        [emb_w, emb_b[:, None], jnp.zeros((V, RW - D - 1), jnp.float32)],
        axis=1)

    W1r = W1.T.reshape(F, D, H1)
    W1p = jnp.concatenate(
        [W1r, jnp.zeros((F, RW - D, H1), jnp.float32)],
        axis=1).reshape(F * RW, H1)
    mcol = jnp.tile(
        jnp.array([0.0] * D + [1.0] + [0.0] * (RW - D - 1),
                  jnp.float32), F)[:, None]
    Wcat = jnp.concatenate([W1p, mcol], axis=1)

    BM = 1024
    Bs = B // _SEG
    Ns = N // _SEG
    outs = []
    for s in range(_SEG):
        fids_s = fids_batch[s * Bs:(s + 1) * Bs].reshape(Ns)
        gathered = _make_gather(Ns, RW, 8)(tab, fids_s)
        X = gathered.reshape(Bs, F * RW)
        out2 = pl.pallas_call(
            _mlp_body,
            grid=(Bs // BM,),
            in_specs=[
                pl.BlockSpec((BM, F * RW), lambda i: (i, 0)),
                pl.BlockSpec((F * RW, H1 + 1), lambda i: (0, 0)),
                pl.BlockSpec((1, H1), lambda i: (0, 0)),
                pl.BlockSpec((H1, H2), lambda i: (0, 0)),
                pl.BlockSpec((1, H2), lambda i: (0, 0)),
                pl.BlockSpec((H2, 1), lambda i: (0, 0)),
                pl.BlockSpec((1, 1), lambda i: (0, 0)),
            ],
            out_specs=pl.BlockSpec((BM, 1), lambda i: (i, 0)),
            out_shape=jax.ShapeDtypeStruct((Bs, 1), jnp.float32),
        )(X, Wcat, b1[None, :], W2.T, b2[None, :], W3.T, b3[None, :])
        outs.append(lax.squeeze(out2, (1,)))

    return jnp.concatenate(outs)
